# baseline (speedup 1.0000x reference)
import functools

import jax
import jax.numpy as jnp
import numpy as np
from jax import lax
from jax.experimental import pallas as pl
from jax.experimental.pallas import tpu as pltpu
from jax.experimental.pallas import tpu_sc as plsc

_N = 10000
_E = 160000
_F = 128
_W = _F // 2
_B = 2
_NS = 16
_EPT = _E // _NS
_C = 16
_NCH = _EPT // _C
_NB = 5
_CR = 624

_PERM = np.concatenate(
    [32 * q + np.concatenate([np.arange(0, 32, 2), np.arange(1, 32, 2)])
     for q in range(4)])


def _widen(u):
    lo = lax.bitcast_convert_type(u << 16, jnp.float32)
    hi = lax.bitcast_convert_type(u & jnp.int32(-65536), jnp.float32)
    return lo, hi


def _narrow_pack(a, b):
    ra = lax.bitcast_convert_type(a, jnp.int32)
    rb = lax.bitcast_convert_type(b, jnp.int32)
    la = ((ra + 0x7FFF + ((ra >> 16) & 1)) >> 16) & 0xFFFF
    lb = ((rb + 0x7FFF + ((rb >> 16) & 1)) >> 16) & 0xFFFF
    return la | (lb << 16)


def _cheb_body(xt_hbm, src_hbm, dst_hbm, wb_hbm, z_hbm, x1_hbm, s1_hbm,
               x1t_hbm,
               srcp, dst_v,
               gb0, gb1, gb2, gb3, gb4, sb0, sb1, sb2, sb3, sb4,
               wv0, wv1, wv2, wv3, wv4, tb, acc,
               gsem0, gsem1, gsem2, gsem3, gsem4,
               ssem0, ssem1, ssem2, ssem3, ssem4):
    c = lax.axis_index("c")
    s = lax.axis_index("s")
    gb = (gb0, gb1, gb2, gb3, gb4)
    sb = (sb0, sb1, sb2, sb3, sb4)
    wv = (wv0, wv1, wv2, wv3, wv4)
    gsem = (gsem0, gsem1, gsem2, gsem3, gsem4)
    ssem = (ssem0, ssem1, ssem2, ssem3, ssem4)

    e0 = pl.multiple_of(s * _EPT, 8)
    pltpu.sync_copy(src_hbm.at[pl.ds(e0, _EPT)], srcp)
    pltpu.sync_copy(dst_hbm.at[pl.ds(e0, _EPT)], dst_v)

    base = c * _N

    def _padd(i, carry):
        o = pl.multiple_of(i * 16, 8)
        srcp[pl.ds(o, 16)] = srcp[pl.ds(o, 16)] + base
        return carry

    lax.fori_loop(0, _EPT // 16, _padd, 0)
    zidx = lax.iota(jnp.int32, 16) * 0

    last_start = (_NS - 1) * _CR
    last_rows = _N - last_start

    def _zero_acc():
        st = pl.multiple_of(s * _CR, 8)

        @pl.when(s < _NS - 1)
        def _():
            pltpu.sync_copy(z_hbm.at[pl.ds(0, _CR)], acc.at[pl.ds(st, _CR)])

        @pl.when(s == _NS - 1)
        def _():
            pltpu.sync_copy(z_hbm.at[pl.ds(0, last_rows)],
                            acc.at[pl.ds(last_start, last_rows)])

    def _copy_out(out_hbm):
        st = pl.multiple_of(s * _CR, 8)
        bb = pl.multiple_of(c * _N, 8)

        @pl.when(s < _NS - 1)
        def _():
            pltpu.sync_copy(acc.at[pl.ds(st, _CR)],
                            out_hbm.at[pl.ds(bb + st, _CR)])

        @pl.when(s == _NS - 1)
        def _():
            pltpu.sync_copy(acc.at[pl.ds(last_start, last_rows)],
                            out_hbm.at[pl.ds(bb + last_start, last_rows)])

    def _copy_out_bf16(out_hbm):
        st = pl.multiple_of(s * _CR, 8)
        bb = pl.multiple_of(c * _N, 8)
        nch = jnp.where(s < _NS - 1, _CR // _C, last_rows // _C)

        def _one(t, carry):
            ro = pl.multiple_of(st + t * _C, 8)
            pltpu.sync_copy(acc.at[pl.ds(ro, _C)], sb0)
            for r in range(_C):
                for q in range(4):
                    a = sb0[r, pl.ds(32 * q, 16)]
                    b = sb0[r, pl.ds(32 * q + 16, 16)]
                    tb[r, pl.ds(16 * q, 16)] = _narrow_pack(a, b)
            pltpu.sync_copy(tb, out_hbm.at[pl.ds(bb + ro, _C)])
            return carry

        lax.fori_loop(0, nch, _one, 0)

    def _hop(table_hbm, out_f32_hbm, out_bf16_hbm):
        def _issue(j, k):
            off = pl.multiple_of(j * _C, 8)
            pltpu.async_copy(table_hbm.at[srcp.at[pl.ds(off, _C)]],
                             gb[k], gsem[k])
            woff = pl.multiple_of((e0 + j * _C) * 16, 8)
            pltpu.async_copy(wb_hbm.at[pl.ds(woff, _C * 16)], wv[k], gsem[k])

        def _wait_gather(k):
            pltpu.make_async_copy(table_hbm.at[pl.ds(0, _C)], gb[k],
                                  gsem[k]).wait()
            pltpu.make_async_copy(wb_hbm.at[pl.ds(0, _C * 16)], wv[k],
                                  gsem[k]).wait()

        def _scale(k):
            for r in range(_C):
                wb16 = wv[k][pl.ds(r * 16, 16)]
                for q in range(4):
                    u = gb[k][r, pl.ds(q * 16, 16)]
                    lo, hi = _widen(u)
                    sb[k][r, pl.ds(32 * q, 16)] = lo * wb16
                    sb[k][r, pl.ds(32 * q + 16, 16)] = hi * wb16

        def _issue_scatter(j, k):
            dv = dst_v[pl.ds(pl.multiple_of(j * _C, 8), _C)]
            pltpu.async_copy(sb[k], acc.at[dv], ssem[k], add=True)

        def _drain_scatter(k):
            pltpu.make_async_copy(z_hbm.at[pl.ds(0, _C)], sb[k],
                                  ssem[k]).wait()

        for k in range(_NB):
            pltpu.sync_copy(z_hbm.at[pl.ds(0, _C)], sb[k])
            pltpu.async_copy(sb[k], acc.at[zidx], ssem[k], add=True)
            _issue(k, k)

        def _body(jj, carry):
            for k in range(_NB):
                j = _NB * jj + k
                _wait_gather(k)
                _drain_scatter(k)
                _scale(k)
                _issue_scatter(j, k)
                _issue(jnp.minimum(j + _NB, _NCH - 1), k)
            return carry

        lax.fori_loop(0, _NCH // _NB, _body, 0)
        for k in range(_NB):
            _wait_gather(k)
            _drain_scatter(k)
        plsc.subcore_barrier()
        _copy_out(out_f32_hbm)
        if out_bf16_hbm is not None:
            _copy_out_bf16(out_bf16_hbm)

    _zero_acc()
    plsc.subcore_barrier()
    _hop(xt_hbm, x1_hbm, x1t_hbm)
    plsc.subcore_barrier()
    _zero_acc()
    plsc.subcore_barrier()
    _hop(x1t_hbm, s1_hbm, None)


_cheb = functools.partial(
    pl.kernel,
    out_type=[jax.ShapeDtypeStruct((_B * _N, _F), jnp.float32),
              jax.ShapeDtypeStruct((_B * _N, _F), jnp.float32),
              jax.ShapeDtypeStruct((_B * _N, _W), jnp.int32)],
    mesh=plsc.VectorSubcoreMesh(core_axis_name="c", subcore_axis_name="s",
                                num_cores=_B, num_subcores=_NS),
    compiler_params=pltpu.CompilerParams(use_tc_tiling_on_sc=False),
    scratch_types=(
        [pltpu.VMEM((_EPT,), jnp.int32),
         pltpu.VMEM((_EPT,), jnp.int32)]
        + [pltpu.VMEM((_C, _W), jnp.int32) for _ in range(_NB)]
        + [pltpu.VMEM((_C, _F), jnp.float32) for _ in range(_NB)]
        + [pltpu.VMEM((_C * 16,), jnp.float32) for _ in range(_NB)]
        + [pltpu.VMEM((_C, _W), jnp.int32)]
        + [pltpu.VMEM_SHARED((_N, _F), jnp.float32)]
        + [pltpu.SemaphoreType.DMA for _ in range(2 * _NB)]
    ),
)(_cheb_body)


_G = 10
_R = (_B * _N) // _G


def _mm0_body(xr, war, br, outr):
    outr[...] = (jnp.dot(xr[...], war[...], preferred_element_type=jnp.float32)
                 + br[...])


def _mm_body(tr, x1r, s1r, wbr, wcr, outr, psr, pqr):
    a = tr[...]
    a = a + jnp.dot(x1r[...], wbr[...], preferred_element_type=jnp.float32)
    a = a + jnp.dot(s1r[...], wcr[...], preferred_element_type=jnp.float32)
    outr[...] = a
    psr[...] = jnp.sum(a, axis=0, keepdims=True).reshape(1, 1, _F)
    pqr[...] = jnp.sum(a * a, axis=0, keepdims=True).reshape(1, 1, _F)


def _fin_body(xr, psr, pqr, gr, betar, outr):
    n = float(_B * _N)
    mean = jnp.sum(psr[...], axis=0) / n
    var = jnp.sum(pqr[...], axis=0) / n - mean * mean
    inv = lax.rsqrt(var + 1e-5)
    y = (xr[...] - mean) * (inv * gr[...]) + betar[...]
    outr[...] = jnp.maximum(y, 0.0)


def kernel(x, edge_index, edge_weight, weight, bias, gamma, beta):
    xflat = x.reshape(_B * _N, _F)
    src = edge_index[0]
    dst = edge_index[1]

    xt = lax.bitcast_convert_type(
        xflat.astype(jnp.bfloat16).reshape(_B * _N, _W, 2), jnp.int32)
    wsplat = jnp.repeat(edge_weight, 16)
    zrows = jnp.zeros((640, _F), jnp.float32)
    x1, s1, _x1t = _cheb(xt, src, dst, wsplat, zrows)

    wr = weight.reshape(_F, 3, _F)
    wa = wr[:, 0, :] - wr[:, 2, :]
    wb = wr[:, 1, :][_PERM, :]
    wc = 2.0 * wr[:, 2, :][_PERM, :]

    tmp = pl.pallas_call(
        _mm0_body,
        grid=(_G,),
        in_specs=[
            pl.BlockSpec((_R, _F), lambda i: (i, 0)),
            pl.BlockSpec((_F, _F), lambda i: (0, 0)),
            pl.BlockSpec((1, _F), lambda i: (0, 0)),
        ],
        out_specs=pl.BlockSpec((_R, _F), lambda i: (i, 0)),
        out_shape=jax.ShapeDtypeStruct((_B * _N, _F), jnp.float32),
    )(xflat, wa, bias.reshape(1, _F))

    out_pre, ps, pq = pl.pallas_call(
        _mm_body,
        grid=(_G,),
        in_specs=[
            pl.BlockSpec((_R, _F), lambda i: (i, 0)),
            pl.BlockSpec((_R, _F), lambda i: (i, 0)),
            pl.BlockSpec((_R, _F), lambda i: (i, 0)),
            pl.BlockSpec((_F, _F), lambda i: (0, 0)),
            pl.BlockSpec((_F, _F), lambda i: (0, 0)),
        ],
        out_specs=[
            pl.BlockSpec((_R, _F), lambda i: (i, 0)),
            pl.BlockSpec((1, 1, _F), lambda i: (i, 0, 0)),
            pl.BlockSpec((1, 1, _F), lambda i: (i, 0, 0)),
        ],
        out_shape=[
            jax.ShapeDtypeStruct((_B * _N, _F), jnp.float32),
            jax.ShapeDtypeStruct((_G, 1, _F), jnp.float32),
            jax.ShapeDtypeStruct((_G, 1, _F), jnp.float32),
        ],
    )(tmp, x1, s1, wb, wc)

    out = pl.pallas_call(
        _fin_body,
        grid=(_G,),
        in_specs=[
            pl.BlockSpec((_R, _F), lambda i: (i, 0)),
            pl.BlockSpec((_G, 1, _F), lambda i: (0, 0, 0)),
            pl.BlockSpec((_G, 1, _F), lambda i: (0, 0, 0)),
            pl.BlockSpec((1, _F), lambda i: (0, 0)),
            pl.BlockSpec((1, _F), lambda i: (0, 0)),
        ],
        out_specs=pl.BlockSpec((_R, _F), lambda i: (i, 0)),
        out_shape=jax.ShapeDtypeStruct((_B * _N, _F), jnp.float32),
    )(out_pre, ps, pq, gamma.reshape(1, _F), beta.reshape(1, _F))

    return out.reshape(_B, _N, _F)


# speedup vs baseline: 1.1748x; 1.1748x over previous
import functools

import jax
import jax.numpy as jnp
import numpy as np
from jax import lax
from jax.experimental import pallas as pl
from jax.experimental.pallas import tpu as pltpu
from jax.experimental.pallas import tpu_sc as plsc

_N = 10000
_E = 160000
_F = 128
_W = _F // 2
_B = 2
_NS = 16
_EPT = _E // _NS
_C = 16
_NCH = _EPT // _C
_NB = 5
_CR = 624

_PERM = np.concatenate(
    [32 * q + np.concatenate([np.arange(0, 32, 2), np.arange(1, 32, 2)])
     for q in range(4)])


def _widen(u):
    lo = lax.bitcast_convert_type(u << 16, jnp.float32)
    hi = lax.bitcast_convert_type(u & jnp.int32(-65536), jnp.float32)
    return lo, hi


def _narrow_pack(a, b):
    ra = lax.bitcast_convert_type(a, jnp.int32)
    rb = lax.bitcast_convert_type(b, jnp.int32)
    la = ((ra + 0x7FFF + ((ra >> 16) & 1)) >> 16) & 0xFFFF
    lb = ((rb + 0x7FFF + ((rb >> 16) & 1)) >> 16) & 0xFFFF
    return la | (lb << 16)


def _cheb_body(x_hbm, src_hbm, dst_hbm, wb_hbm, z_hbm, x1_hbm, s1_hbm,
               srcp, dst_v,
               gb0, gb1, gb2, gb3, gb4, sb0, sb1, sb2, sb3, sb4,
               wv0, wv1, wv2, wv3, wv4, acc,
               gsem0, gsem1, gsem2, gsem3, gsem4,
               ssem0, ssem1, ssem2, ssem3, ssem4):
    c = lax.axis_index("c")
    s = lax.axis_index("s")
    gb = (gb0, gb1, gb2, gb3, gb4)
    sb = (sb0, sb1, sb2, sb3, sb4)
    wv = (wv0, wv1, wv2, wv3, wv4)
    gsem = (gsem0, gsem1, gsem2, gsem3, gsem4)
    ssem = (ssem0, ssem1, ssem2, ssem3, ssem4)

    e0 = pl.multiple_of(s * _EPT, 8)
    pltpu.sync_copy(src_hbm.at[pl.ds(e0, _EPT)], srcp)
    pltpu.sync_copy(dst_hbm.at[pl.ds(e0, _EPT)], dst_v)

    base = c * _N

    def _padd(i, carry):
        o = pl.multiple_of(i * 16, 8)
        srcp[pl.ds(o, 16)] = srcp[pl.ds(o, 16)] + base
        return carry

    lax.fori_loop(0, _EPT // 16, _padd, 0)
    zidx = lax.iota(jnp.int32, 16) * 0

    last_start = (_NS - 1) * _CR
    last_rows = _N - last_start

    def _zero_acc():
        st = pl.multiple_of(s * _CR, 8)

        @pl.when(s < _NS - 1)
        def _():
            pltpu.sync_copy(z_hbm.at[pl.ds(0, _CR)], acc.at[pl.ds(st, _CR)])

        @pl.when(s == _NS - 1)
        def _():
            pltpu.sync_copy(z_hbm.at[pl.ds(0, last_rows)],
                            acc.at[pl.ds(last_start, last_rows)])

    def _copy_out(out_hbm):
        st = pl.multiple_of(s * _CR, 8)
        bb = pl.multiple_of(c * _N, 8)

        @pl.when(s < _NS - 1)
        def _():
            pltpu.sync_copy(acc.at[pl.ds(st, _CR)],
                            out_hbm.at[pl.ds(bb + st, _CR)])

        @pl.when(s == _NS - 1)
        def _():
            pltpu.sync_copy(acc.at[pl.ds(last_start, last_rows)],
                            out_hbm.at[pl.ds(bb + last_start, last_rows)])

    def _hop(table_hbm, out_f32_hbm):
        def _issue(j, k):
            off = pl.multiple_of(j * _C, 8)
            pltpu.async_copy(table_hbm.at[srcp.at[pl.ds(off, _C)]],
                             gb[k], gsem[k])
            woff = pl.multiple_of((e0 + j * _C) * 16, 8)
            pltpu.async_copy(wb_hbm.at[pl.ds(woff, _C * 16)], wv[k], gsem[k])

        def _wait_gather(k):
            pltpu.make_async_copy(table_hbm.at[pl.ds(0, _C)], gb[k],
                                  gsem[k]).wait()
            pltpu.make_async_copy(wb_hbm.at[pl.ds(0, _C * 16)], wv[k],
                                  gsem[k]).wait()

        def _scale(k):
            for r in range(_C):
                wb16 = wv[k][pl.ds(r * 16, 16)]
                for q in range(_F // 16):
                    sb[k][r, pl.ds(q * 16, 16)] = (
                        gb[k][r, pl.ds(q * 16, 16)] * wb16)

        def _issue_scatter(j, k):
            dv = dst_v[pl.ds(pl.multiple_of(j * _C, 8), _C)]
            pltpu.async_copy(sb[k], acc.at[dv], ssem[k], add=True)

        def _drain_scatter(k):
            pltpu.make_async_copy(z_hbm.at[pl.ds(0, _C)], sb[k],
                                  ssem[k]).wait()

        for k in range(_NB):
            pltpu.sync_copy(z_hbm.at[pl.ds(0, _C)], sb[k])
            pltpu.async_copy(sb[k], acc.at[zidx], ssem[k], add=True)
            _issue(k, k)

        def _body(jj, carry):
            for k in range(_NB):
                j = _NB * jj + k
                _wait_gather(k)
                _drain_scatter(k)
                _scale(k)
                _issue_scatter(j, k)
                _issue(jnp.minimum(j + _NB, _NCH - 1), k)
            return carry

        lax.fori_loop(0, _NCH // _NB, _body, 0)
        for k in range(_NB):
            _wait_gather(k)
            _drain_scatter(k)
        plsc.subcore_barrier()
        _copy_out(out_f32_hbm)

    _zero_acc()
    plsc.subcore_barrier()
    _hop(x_hbm, x1_hbm)
    plsc.subcore_barrier()
    _zero_acc()
    plsc.subcore_barrier()
    _hop(x1_hbm, s1_hbm)


_cheb = functools.partial(
    pl.kernel,
    out_type=[jax.ShapeDtypeStruct((_B * _N, _F), jnp.float32),
              jax.ShapeDtypeStruct((_B * _N, _F), jnp.float32)],
    mesh=plsc.VectorSubcoreMesh(core_axis_name="c", subcore_axis_name="s",
                                num_cores=_B, num_subcores=_NS),
    scratch_types=(
        [pltpu.VMEM((_EPT,), jnp.int32),
         pltpu.VMEM((_EPT,), jnp.int32)]
        + [pltpu.VMEM((_C, _F), jnp.float32) for _ in range(_NB)]
        + [pltpu.VMEM((_C, _F), jnp.float32) for _ in range(_NB)]
        + [pltpu.VMEM((_C * 16,), jnp.float32) for _ in range(_NB)]
        + [pltpu.VMEM_SHARED((_N, _F), jnp.float32)]
        + [pltpu.SemaphoreType.DMA for _ in range(2 * _NB)]
    ),
)(_cheb_body)


_G = 10
_R = (_B * _N) // _G


def _mm0_body(xr, war, br, outr):
    outr[...] = (jnp.dot(xr[...], war[...], preferred_element_type=jnp.float32)
                 + br[...])


def _mm_body(tr, x1r, s1r, wbr, wcr, outr, psr, pqr):
    a = tr[...]
    a = a + jnp.dot(x1r[...], wbr[...], preferred_element_type=jnp.float32)
    a = a + jnp.dot(s1r[...], wcr[...], preferred_element_type=jnp.float32)
    outr[...] = a
    psr[...] = jnp.sum(a, axis=0, keepdims=True).reshape(1, 1, _F)
    pqr[...] = jnp.sum(a * a, axis=0, keepdims=True).reshape(1, 1, _F)


def _fin_body(xr, psr, pqr, gr, betar, outr):
    n = float(_B * _N)
    mean = jnp.sum(psr[...], axis=0) / n
    var = jnp.sum(pqr[...], axis=0) / n - mean * mean
    inv = lax.rsqrt(var + 1e-5)
    y = (xr[...] - mean) * (inv * gr[...]) + betar[...]
    outr[...] = jnp.maximum(y, 0.0)


def kernel(x, edge_index, edge_weight, weight, bias, gamma, beta):
    xflat = x.reshape(_B * _N, _F)
    src = edge_index[0]
    dst = edge_index[1]

    wsplat = jnp.repeat(edge_weight, 16)
    zrows = jnp.zeros((640, _F), jnp.float32)
    x1, s1 = _cheb(xflat, src, dst, wsplat, zrows)

    wr = weight.reshape(_F, 3, _F)
    wa = wr[:, 0, :] - wr[:, 2, :]
    wb = wr[:, 1, :]
    wc = 2.0 * wr[:, 2, :]

    tmp = pl.pallas_call(
        _mm0_body,
        grid=(_G,),
        in_specs=[
            pl.BlockSpec((_R, _F), lambda i: (i, 0)),
            pl.BlockSpec((_F, _F), lambda i: (0, 0)),
            pl.BlockSpec((1, _F), lambda i: (0, 0)),
        ],
        out_specs=pl.BlockSpec((_R, _F), lambda i: (i, 0)),
        out_shape=jax.ShapeDtypeStruct((_B * _N, _F), jnp.float32),
    )(xflat, wa, bias.reshape(1, _F))

    out_pre, ps, pq = pl.pallas_call(
        _mm_body,
        grid=(_G,),
        in_specs=[
            pl.BlockSpec((_R, _F), lambda i: (i, 0)),
            pl.BlockSpec((_R, _F), lambda i: (i, 0)),
            pl.BlockSpec((_R, _F), lambda i: (i, 0)),
            pl.BlockSpec((_F, _F), lambda i: (0, 0)),
            pl.BlockSpec((_F, _F), lambda i: (0, 0)),
        ],
        out_specs=[
            pl.BlockSpec((_R, _F), lambda i: (i, 0)),
            pl.BlockSpec((1, 1, _F), lambda i: (i, 0, 0)),
            pl.BlockSpec((1, 1, _F), lambda i: (i, 0, 0)),
        ],
        out_shape=[
            jax.ShapeDtypeStruct((_B * _N, _F), jnp.float32),
            jax.ShapeDtypeStruct((_G, 1, _F), jnp.float32),
            jax.ShapeDtypeStruct((_G, 1, _F), jnp.float32),
        ],
    )(tmp, x1, s1, wb, wc)

    out = pl.pallas_call(
        _fin_body,
        grid=(_G,),
        in_specs=[
            pl.BlockSpec((_R, _F), lambda i: (i, 0)),
            pl.BlockSpec((_G, 1, _F), lambda i: (0, 0, 0)),
            pl.BlockSpec((_G, 1, _F), lambda i: (0, 0, 0)),
            pl.BlockSpec((1, _F), lambda i: (0, 0)),
            pl.BlockSpec((1, _F), lambda i: (0, 0)),
        ],
        out_specs=pl.BlockSpec((_R, _F), lambda i: (i, 0)),
        out_shape=jax.ShapeDtypeStruct((_B * _N, _F), jnp.float32),
    )(out_pre, ps, pq, gamma.reshape(1, _F), beta.reshape(1, _F))

    return out.reshape(_B, _N, _F)


# speedup vs baseline: 1.1834x; 1.0073x over previous
import functools

import jax
import jax.numpy as jnp
import numpy as np
from jax import lax
from jax.experimental import pallas as pl
from jax.experimental.pallas import tpu as pltpu
from jax.experimental.pallas import tpu_sc as plsc

_N = 10000
_E = 160000
_F = 128
_W = _F // 2
_B = 2
_NS = 16
_EPT = _E // _NS
_C = 16
_NCH = _EPT // _C
_NB = 5
_CR = 624

_PERM = np.concatenate(
    [32 * q + np.concatenate([np.arange(0, 32, 2), np.arange(1, 32, 2)])
     for q in range(4)])


def _widen(u):
    lo = lax.bitcast_convert_type(u << 16, jnp.float32)
    hi = lax.bitcast_convert_type(u & jnp.int32(-65536), jnp.float32)
    return lo, hi


def _narrow_pack(a, b):
    ra = lax.bitcast_convert_type(a, jnp.int32)
    rb = lax.bitcast_convert_type(b, jnp.int32)
    la = ((ra + 0x7FFF + ((ra >> 16) & 1)) >> 16) & 0xFFFF
    lb = ((rb + 0x7FFF + ((rb >> 16) & 1)) >> 16) & 0xFFFF
    return la | (lb << 16)


def _cheb_body(x_hbm, src_hbm, dst_hbm, wb_hbm, z_hbm, x1_hbm, s1_hbm,
               srcp, dst_v,
               gb0, gb1, gb2, gb3, gb4, sb0, sb1, sb2, sb3, sb4,
               wv0, wv1, wv2, wv3, wv4, acc,
               gsem0, gsem1, gsem2, gsem3, gsem4,
               ssem0, ssem1, ssem2, ssem3, ssem4):
    c = lax.axis_index("c")
    s = lax.axis_index("s")
    gb = (gb0, gb1, gb2, gb3, gb4)
    sb = (sb0, sb1, sb2, sb3, sb4)
    wv = (wv0, wv1, wv2, wv3, wv4)
    gsem = (gsem0, gsem1, gsem2, gsem3, gsem4)
    ssem = (ssem0, ssem1, ssem2, ssem3, ssem4)

    e0 = pl.multiple_of(s * _EPT, 8)
    pltpu.sync_copy(src_hbm.at[pl.ds(e0, _EPT)], srcp)
    pltpu.sync_copy(dst_hbm.at[pl.ds(e0, _EPT)], dst_v)

    base = c * _N

    def _padd(i, carry):
        o = pl.multiple_of(i * 16, 8)
        srcp[pl.ds(o, 16)] = srcp[pl.ds(o, 16)] + base
        return carry

    lax.fori_loop(0, _EPT // 16, _padd, 0)
    zidx = lax.iota(jnp.int32, 16) * 0

    last_start = (_NS - 1) * _CR
    last_rows = _N - last_start

    def _zero_acc():
        st = pl.multiple_of(s * _CR, 8)

        @pl.when(s < _NS - 1)
        def _():
            pltpu.sync_copy(z_hbm.at[pl.ds(0, _CR)], acc.at[pl.ds(st, _CR)])

        @pl.when(s == _NS - 1)
        def _():
            pltpu.sync_copy(z_hbm.at[pl.ds(0, last_rows)],
                            acc.at[pl.ds(last_start, last_rows)])

    def _copy_out(out_hbm):
        st = pl.multiple_of(s * _CR, 8)
        bb = pl.multiple_of(c * _N, 8)

        @pl.when(s < _NS - 1)
        def _():
            pltpu.sync_copy(acc.at[pl.ds(st, _CR)],
                            out_hbm.at[pl.ds(bb + st, _CR)])

        @pl.when(s == _NS - 1)
        def _():
            pltpu.sync_copy(acc.at[pl.ds(last_start, last_rows)],
                            out_hbm.at[pl.ds(bb + last_start, last_rows)])

    def _hop(table_hbm, out_f32_hbm):
        def _issue(j, k):
            off = pl.multiple_of(j * _C, 8)
            pltpu.async_copy(table_hbm.at[srcp.at[pl.ds(off, _C)]],
                             gb[k], gsem[k])
            woff = pl.multiple_of((e0 + j * _C) * 16, 8)
            pltpu.async_copy(wb_hbm.at[pl.ds(woff, _C * 16)], wv[k], gsem[k])

        def _wait_gather(k):
            pltpu.make_async_copy(table_hbm.at[pl.ds(0, _C)], gb[k],
                                  gsem[k]).wait()
            pltpu.make_async_copy(wb_hbm.at[pl.ds(0, _C * 16)], wv[k],
                                  gsem[k]).wait()

        def _scale(k):
            for r in range(_C):
                wb16 = wv[k][pl.ds(r * 16, 16)]
                for q in range(_F // 16):
                    sb[k][r, pl.ds(q * 16, 16)] = (
                        gb[k][r, pl.ds(q * 16, 16)] * wb16)

        def _issue_scatter(j, k):
            dv = dst_v[pl.ds(pl.multiple_of(j * _C, 8), _C)]
            pltpu.async_copy(sb[k], acc.at[dv], ssem[k], add=True)

        def _drain_scatter(k):
            pltpu.make_async_copy(z_hbm.at[pl.ds(0, _C)], sb[k],
                                  ssem[k]).wait()

        for k in range(_NB):
            pltpu.sync_copy(z_hbm.at[pl.ds(0, _C)], sb[k])
            pltpu.async_copy(sb[k], acc.at[zidx], ssem[k], add=True)
            _issue(k, k)

        def _body(jj, carry):
            for k in range(_NB):
                j = _NB * jj + k
                _wait_gather(k)
                _drain_scatter(k)
                _scale(k)
                _issue_scatter(j, k)
                _issue(jnp.minimum(j + _NB, _NCH - 1), k)
            return carry

        lax.fori_loop(0, _NCH // _NB, _body, 0)
        for k in range(_NB):
            _wait_gather(k)
            _drain_scatter(k)
        plsc.subcore_barrier()
        _copy_out(out_f32_hbm)

    _zero_acc()
    plsc.subcore_barrier()
    _hop(x_hbm, x1_hbm)
    plsc.subcore_barrier()
    _zero_acc()
    plsc.subcore_barrier()
    _hop(x1_hbm, s1_hbm)


_cheb = functools.partial(
    pl.kernel,
    out_type=[jax.ShapeDtypeStruct((_B * _N, _F), jnp.float32),
              jax.ShapeDtypeStruct((_B * _N, _F), jnp.float32)],
    mesh=plsc.VectorSubcoreMesh(core_axis_name="c", subcore_axis_name="s",
                                num_cores=_B, num_subcores=_NS),
    scratch_types=(
        [pltpu.VMEM((_EPT,), jnp.int32),
         pltpu.VMEM((_EPT,), jnp.int32)]
        + [pltpu.VMEM((_C, _F), jnp.float32) for _ in range(_NB)]
        + [pltpu.VMEM((_C, _F), jnp.float32) for _ in range(_NB)]
        + [pltpu.VMEM((_C * 16,), jnp.float32) for _ in range(_NB)]
        + [pltpu.VMEM_SHARED((_N, _F), jnp.float32)]
        + [pltpu.SemaphoreType.DMA for _ in range(2 * _NB)]
    ),
)(_cheb_body)


_G = 10
_R = (_B * _N) // _G


def _mm_body(xr, x1r, s1r, war, wbr, wcr, br, outr, psr, pqr):
    a = jnp.dot(xr[...], war[...], preferred_element_type=jnp.float32)
    a = a + jnp.dot(x1r[...], wbr[...], preferred_element_type=jnp.float32)
    a = a + jnp.dot(s1r[...], wcr[...], preferred_element_type=jnp.float32)
    a = a + br[...]
    outr[...] = a
    psr[...] = jnp.sum(a, axis=0, keepdims=True).reshape(1, 1, _F)
    pqr[...] = jnp.sum(a * a, axis=0, keepdims=True).reshape(1, 1, _F)


def _fin_body(xr, psr, pqr, gr, betar, outr):
    n = float(_B * _N)
    mean = jnp.sum(psr[...], axis=0) / n
    var = jnp.sum(pqr[...], axis=0) / n - mean * mean
    inv = lax.rsqrt(var + 1e-5)
    y = (xr[...] - mean) * (inv * gr[...]) + betar[...]
    outr[...] = jnp.maximum(y, 0.0)


def kernel(x, edge_index, edge_weight, weight, bias, gamma, beta):
    xflat = x.reshape(_B * _N, _F)
    src = edge_index[0]
    dst = edge_index[1]

    wsplat = jnp.repeat(edge_weight, 16)
    zrows = jnp.zeros((640, _F), jnp.float32)
    x1, s1 = _cheb(xflat, src, dst, wsplat, zrows)

    wr = weight.reshape(_F, 3, _F)
    wa = wr[:, 0, :] - wr[:, 2, :]
    wb = wr[:, 1, :]
    wc = 2.0 * wr[:, 2, :]

    out_pre, ps, pq = pl.pallas_call(
        _mm_body,
        grid=(_G,),
        in_specs=[
            pl.BlockSpec((_R, _F), lambda i: (i, 0)),
            pl.BlockSpec((_R, _F), lambda i: (i, 0)),
            pl.BlockSpec((_R, _F), lambda i: (i, 0)),
            pl.BlockSpec((_F, _F), lambda i: (0, 0)),
            pl.BlockSpec((_F, _F), lambda i: (0, 0)),
            pl.BlockSpec((_F, _F), lambda i: (0, 0)),
            pl.BlockSpec((1, _F), lambda i: (0, 0)),
        ],
        out_specs=[
            pl.BlockSpec((_R, _F), lambda i: (i, 0)),
            pl.BlockSpec((1, 1, _F), lambda i: (i, 0, 0)),
            pl.BlockSpec((1, 1, _F), lambda i: (i, 0, 0)),
        ],
        out_shape=[
            jax.ShapeDtypeStruct((_B * _N, _F), jnp.float32),
            jax.ShapeDtypeStruct((_G, 1, _F), jnp.float32),
            jax.ShapeDtypeStruct((_G, 1, _F), jnp.float32),
        ],
    )(xflat, x1, s1, wa, wb, wc, bias.reshape(1, _F))

    out = pl.pallas_call(
        _fin_body,
        grid=(_G,),
        in_specs=[
            pl.BlockSpec((_R, _F), lambda i: (i, 0)),
            pl.BlockSpec((_G, 1, _F), lambda i: (0, 0, 0)),
            pl.BlockSpec((_G, 1, _F), lambda i: (0, 0, 0)),
            pl.BlockSpec((1, _F), lambda i: (0, 0)),
            pl.BlockSpec((1, _F), lambda i: (0, 0)),
        ],
        out_specs=pl.BlockSpec((_R, _F), lambda i: (i, 0)),
        out_shape=jax.ShapeDtypeStruct((_B * _N, _F), jnp.float32),
    )(out_pre, ps, pq, gamma.reshape(1, _F), beta.reshape(1, _F))

    return out.reshape(_B, _N, _F)


# speedup vs baseline: 1.2003x; 1.0143x over previous
import functools

import jax
import jax.numpy as jnp
import numpy as np
from jax import lax
from jax.experimental import pallas as pl
from jax.experimental.pallas import tpu as pltpu
from jax.experimental.pallas import tpu_sc as plsc

_N = 10000
_E = 160000
_F = 128
_W = _F // 2
_B = 2
_NS = 16
_EPT = _E // _NS
_C = 16
_NCH = _EPT // _C
_NB = 5
_CR = 624

_PERM = np.concatenate(
    [32 * q + np.concatenate([np.arange(0, 32, 2), np.arange(1, 32, 2)])
     for q in range(4)])


def _widen(u):
    lo = lax.bitcast_convert_type(u << 16, jnp.float32)
    hi = lax.bitcast_convert_type(u & jnp.int32(-65536), jnp.float32)
    return lo, hi


def _narrow_pack(a, b):
    ra = lax.bitcast_convert_type(a, jnp.int32)
    rb = lax.bitcast_convert_type(b, jnp.int32)
    la = ((ra + 0x7FFF + ((ra >> 16) & 1)) >> 16) & 0xFFFF
    lb = ((rb + 0x7FFF + ((rb >> 16) & 1)) >> 16) & 0xFFFF
    return la | (lb << 16)


def _cheb_body(x_hbm, src_hbm, dst_hbm, wb_hbm, z_hbm, x1_hbm, s1_hbm,
               srcp, dst_v,
               gb0, gb1, gb2, gb3, gb4, sb0, sb1, sb2, sb3, sb4,
               wv0, wv1, wv2, wv3, wv4, acc,
               gsem0, gsem1, gsem2, gsem3, gsem4,
               ssem0, ssem1, ssem2, ssem3, ssem4):
    c = lax.axis_index("c")
    s = lax.axis_index("s")
    gb = (gb0, gb1, gb2, gb3, gb4)
    sb = (sb0, sb1, sb2, sb3, sb4)
    wv = (wv0, wv1, wv2, wv3, wv4)
    gsem = (gsem0, gsem1, gsem2, gsem3, gsem4)
    ssem = (ssem0, ssem1, ssem2, ssem3, ssem4)

    e0 = pl.multiple_of(s * _EPT, 8)
    pltpu.sync_copy(src_hbm.at[pl.ds(e0, _EPT)], srcp)
    pltpu.sync_copy(dst_hbm.at[pl.ds(e0, _EPT)], dst_v)

    base = c * _N

    def _padd(i, carry):
        o = pl.multiple_of(i * 16, 8)
        srcp[pl.ds(o, 16)] = srcp[pl.ds(o, 16)] + base
        return carry

    lax.fori_loop(0, _EPT // 16, _padd, 0)
    zidx = lax.iota(jnp.int32, 16) * 0

    last_start = (_NS - 1) * _CR
    last_rows = _N - last_start

    def _zero_acc():
        st = pl.multiple_of(s * _CR, 8)

        @pl.when(s < _NS - 1)
        def _():
            pltpu.sync_copy(z_hbm.at[pl.ds(st, _CR)], acc.at[pl.ds(st, _CR)])

        @pl.when(s == _NS - 1)
        def _():
            pltpu.sync_copy(z_hbm.at[pl.ds(last_start, last_rows)],
                            acc.at[pl.ds(last_start, last_rows)])

    def _copy_out(out_hbm):
        st = pl.multiple_of(s * _CR, 8)
        bb = pl.multiple_of(c * _N, 8)

        @pl.when(s < _NS - 1)
        def _():
            pltpu.sync_copy(acc.at[pl.ds(st, _CR)],
                            out_hbm.at[pl.ds(bb + st, _CR)])

        @pl.when(s == _NS - 1)
        def _():
            pltpu.sync_copy(acc.at[pl.ds(last_start, last_rows)],
                            out_hbm.at[pl.ds(bb + last_start, last_rows)])

    def _hop(table_hbm, out_f32_hbm):
        def _issue(j, k):
            off = pl.multiple_of(j * _C, 8)
            pltpu.async_copy(table_hbm.at[srcp.at[pl.ds(off, _C)]],
                             gb[k], gsem[k])
            woff = pl.multiple_of((e0 + j * _C) * 16, 8)
            pltpu.async_copy(wb_hbm.at[pl.ds(woff, _C * 16)], wv[k], gsem[k])

        def _wait_gather(k):
            pltpu.make_async_copy(table_hbm.at[pl.ds(0, _C)], gb[k],
                                  gsem[k]).wait()
            pltpu.make_async_copy(wb_hbm.at[pl.ds(0, _C * 16)], wv[k],
                                  gsem[k]).wait()

        def _scale(k):
            for r in range(_C):
                wb16 = wv[k][pl.ds(r * 16, 16)]
                for q in range(_F // 16):
                    sb[k][r, pl.ds(q * 16, 16)] = (
                        gb[k][r, pl.ds(q * 16, 16)] * wb16)

        def _issue_scatter(j, k):
            dv = dst_v[pl.ds(pl.multiple_of(j * _C, 8), _C)]
            pltpu.async_copy(sb[k], acc.at[dv], ssem[k], add=True)

        def _drain_scatter(k):
            pltpu.make_async_copy(z_hbm.at[pl.ds(0, _C)], sb[k],
                                  ssem[k]).wait()

        for k in range(_NB):
            pltpu.sync_copy(z_hbm.at[pl.ds(0, _C)], sb[k])
            pltpu.async_copy(sb[k], acc.at[zidx], ssem[k], add=True)
            _issue(k, k)

        def _body(jj, carry):
            for k in range(_NB):
                j = _NB * jj + k
                _wait_gather(k)
                _drain_scatter(k)
                _scale(k)
                _issue_scatter(j, k)
                _issue(jnp.minimum(j + _NB, _NCH - 1), k)
            return carry

        lax.fori_loop(0, _NCH // _NB, _body, 0)
        for k in range(_NB):
            _wait_gather(k)
            _drain_scatter(k)
        plsc.subcore_barrier()
        _copy_out(out_f32_hbm)

    _zero_acc()
    plsc.subcore_barrier()
    _hop(x_hbm, x1_hbm)
    plsc.subcore_barrier()
    _zero_acc()
    plsc.subcore_barrier()
    _hop(x1_hbm, s1_hbm)


_cheb = functools.partial(
    pl.kernel,
    out_type=[jax.ShapeDtypeStruct((_B * _N, _F), jnp.float32),
              jax.ShapeDtypeStruct((_B * _N, _F), jnp.float32)],
    mesh=plsc.VectorSubcoreMesh(core_axis_name="c", subcore_axis_name="s",
                                num_cores=_B, num_subcores=_NS),
    scratch_types=(
        [pltpu.VMEM((_EPT,), jnp.int32),
         pltpu.VMEM((_EPT,), jnp.int32)]
        + [pltpu.VMEM((_C, _F), jnp.float32) for _ in range(_NB)]
        + [pltpu.VMEM((_C, _F), jnp.float32) for _ in range(_NB)]
        + [pltpu.VMEM((_C * 16,), jnp.float32) for _ in range(_NB)]
        + [pltpu.VMEM_SHARED((_N, _F), jnp.float32)]
        + [pltpu.SemaphoreType.DMA for _ in range(2 * _NB)]
    ),
)(_cheb_body)


_G = 10
_R = (_B * _N) // _G


def _fused_body(xr, x1r, s1r, war, wbr, wcr, br, gr, betar, outr,
                buf, ps, pq):
    p = pl.program_id(0)
    i = pl.program_id(1)
    n = float(_B * _N)

    @pl.when(p == 0)
    def _():
        a = jnp.dot(xr[...], war[...], preferred_element_type=jnp.float32)
        a = a + jnp.dot(x1r[...], wbr[...], preferred_element_type=jnp.float32)
        a = a + jnp.dot(s1r[...], wcr[...], preferred_element_type=jnp.float32)
        a = a + br[...]
        buf[pl.ds(i * _R, _R), :] = a

        @pl.when(i == 0)
        def _():
            ps[...] = jnp.zeros((1, _F), jnp.float32)
            pq[...] = jnp.zeros((1, _F), jnp.float32)

        ps[...] += jnp.sum(a, axis=0, keepdims=True)
        pq[...] += jnp.sum(a * a, axis=0, keepdims=True)

    @pl.when(p == 1)
    def _():
        mean = ps[...] / n
        var = pq[...] / n - mean * mean
        inv = lax.rsqrt(var + 1e-5)
        y = (buf[pl.ds(i * _R, _R), :] - mean) * (inv * gr[...]) + betar[...]
        outr[...] = jnp.maximum(y, 0.0)


def kernel(x, edge_index, edge_weight, weight, bias, gamma, beta):
    xflat = x.reshape(_B * _N, _F)
    src = edge_index[0]
    dst = edge_index[1]

    wsplat = jnp.repeat(edge_weight, 16)
    zrows = jnp.zeros((_N, _F), jnp.float32)
    x1, s1 = _cheb(xflat, src, dst, wsplat, zrows)

    wr = weight.reshape(_F, 3, _F)
    wa = wr[:, 0, :] - wr[:, 2, :]
    wb = wr[:, 1, :]
    wc = 2.0 * wr[:, 2, :]

    out = pl.pallas_call(
        _fused_body,
        grid=(2, _G),
        in_specs=[
            pl.BlockSpec((_R, _F), lambda p, i: (i * (1 - p), 0)),
            pl.BlockSpec((_R, _F), lambda p, i: (i * (1 - p), 0)),
            pl.BlockSpec((_R, _F), lambda p, i: (i * (1 - p), 0)),
            pl.BlockSpec((_F, _F), lambda p, i: (0, 0)),
            pl.BlockSpec((_F, _F), lambda p, i: (0, 0)),
            pl.BlockSpec((_F, _F), lambda p, i: (0, 0)),
            pl.BlockSpec((1, _F), lambda p, i: (0, 0)),
            pl.BlockSpec((1, _F), lambda p, i: (0, 0)),
            pl.BlockSpec((1, _F), lambda p, i: (0, 0)),
        ],
        out_specs=pl.BlockSpec((_R, _F), lambda p, i: (i * p, 0)),
        out_shape=jax.ShapeDtypeStruct((_B * _N, _F), jnp.float32),
        scratch_shapes=[
            pltpu.VMEM((_B * _N, _F), jnp.float32),
            pltpu.VMEM((1, _F), jnp.float32),
            pltpu.VMEM((1, _F), jnp.float32),
        ],
    )(xflat, x1, s1, wa, wb, wc, bias.reshape(1, _F),
      gamma.reshape(1, _F), beta.reshape(1, _F))

    return out.reshape(_B, _N, _F)
